# baseline (speedup 1.0000x reference)
import functools

import jax
import jax.numpy as jnp
from jax import lax
from jax.experimental import pallas as pl
from jax.experimental.pallas import tpu as pltpu
from jax.experimental.pallas import tpu_sc as plsc

_LANES = 16

_K1 = 0
_K2 = 123
_K3 = _K1 ^ _K2 ^ 0x1BD11BDA
_ROT0 = (13, 15, 26, 6)
_ROT1 = (17, 29, 16, 24)


def _rotl(x, r):
    return (x << jnp.uint32(r)) | (x >> jnp.uint32(32 - r))


def _threefry2x32(x0, x1):
    ks = (jnp.uint32(_K1), jnp.uint32(_K2), jnp.uint32(_K3))
    x0 = x0 + ks[0]
    x1 = x1 + ks[1]

    def rounds(x0, x1, rots):
        for r in rots:
            x0 = x0 + x1
            x1 = x0 ^ _rotl(x1, r)
        return x0, x1

    x0, x1 = rounds(x0, x1, _ROT0)
    x0, x1 = x0 + ks[1], x1 + ks[2] + jnp.uint32(1)
    x0, x1 = rounds(x0, x1, _ROT1)
    x0, x1 = x0 + ks[2], x1 + ks[0] + jnp.uint32(2)
    x0, x1 = rounds(x0, x1, _ROT0)
    x0, x1 = x0 + ks[0], x1 + ks[1] + jnp.uint32(3)
    x0, x1 = rounds(x0, x1, _ROT1)
    x0, x1 = x0 + ks[1], x1 + ks[2] + jnp.uint32(4)
    x0, x1 = rounds(x0, x1, _ROT0)
    x0, x1 = x0 + ks[2], x1 + ks[0] + jnp.uint32(5)
    return x0, x1


def _sc_sampler(num_cores, rows_per_worker, num_actions,
                action_hbm, action_v):
    wid = lax.axis_index("s") * num_cores + lax.axis_index("c")
    row_base = wid * rows_per_worker

    lane = lax.iota(jnp.int32, _LANES)
    zero_u32 = jnp.zeros((_LANES,), jnp.uint32)

    for g in range(rows_per_worker // _LANES):
        rows_u32 = (lane + jnp.int32(row_base + g * _LANES)).astype(jnp.uint32)

        def step(a, carry):
            best_bits, best_act = carry
            cnt = rows_u32 * jnp.uint32(num_actions) + a.astype(jnp.uint32)
            b0, b1 = _threefry2x32(zero_u32, cnt)
            key23 = (b0 ^ b1) >> jnp.uint32(9)
            take = key23 > best_bits
            best_bits = jnp.where(take, key23, best_bits)
            best_act = jnp.where(take, jnp.broadcast_to(a, (_LANES,)), best_act)
            return best_bits, best_act

        _, best_act = lax.fori_loop(
            0, num_actions, step,
            (jnp.zeros((_LANES,), jnp.uint32), jnp.zeros((_LANES,), jnp.int32)))
        action_v[pl.ds(g * _LANES, _LANES)] = best_act

    pltpu.sync_copy(action_v, action_hbm.at[pl.ds(row_base, rows_per_worker)])


def _tc_dense(theta_ref, logits_ref, baseline_ref):
    z = theta_ref[...] * 0.0
    t, b, a = logits_ref.shape
    logits_ref[...] = jnp.broadcast_to(z[None, None, :], (t, b, a))
    baseline_ref[...] = jnp.broadcast_to(jnp.sum(z), (t, b))


def kernel(observation, theta, core_state):
    T, B = observation.shape[0], observation.shape[1]
    A = theta.shape[0]
    n_rows = T * B

    info = plsc.get_sparse_core_info()
    num_cores, num_subcores = info.num_cores, info.num_subcores
    num_workers = num_cores * num_subcores
    assert n_rows % (num_workers * _LANES) == 0
    rows_per_worker = n_rows // num_workers

    mesh = plsc.VectorSubcoreMesh(
        core_axis_name="c", subcore_axis_name="s", num_cores=num_cores)
    sample = pl.kernel(
        functools.partial(_sc_sampler, num_cores, rows_per_worker, A),
        out_type=jax.ShapeDtypeStruct((n_rows,), jnp.int32),
        mesh=mesh,
        scratch_types=(pltpu.VMEM((rows_per_worker,), jnp.int32),),
    )

    dense = pl.pallas_call(
        _tc_dense,
        out_shape=(
            jax.ShapeDtypeStruct((T, B, A), jnp.float32),
            jax.ShapeDtypeStruct((T, B), jnp.float32),
        ),
    )

    action_flat = sample()
    policy_logits, baseline = dense(theta)
    action = action_flat.reshape(T, B)
    return (policy_logits, baseline, action)


# speedup vs baseline: 1.0499x; 1.0499x over previous
import functools

import jax
import jax.numpy as jnp
from jax import lax
from jax.experimental import pallas as pl
from jax.experimental.pallas import tpu as pltpu
from jax.experimental.pallas import tpu_sc as plsc

_LANES = 16

_K1 = 0
_K2 = 123
_K3 = _K1 ^ _K2 ^ 0x1BD11BDA
_ROT0 = (13, 15, 26, 6)
_ROT1 = (17, 29, 16, 24)


def _rotl(x, r):
    return (x << jnp.uint32(r)) | (x >> jnp.uint32(32 - r))


def _threefry2x32(x0, x1):
    ks = (jnp.uint32(_K1), jnp.uint32(_K2), jnp.uint32(_K3))
    x0 = x0 + ks[0]
    x1 = x1 + ks[1]

    def rounds(x0, x1, rots):
        for r in rots:
            x0 = x0 + x1
            x1 = x0 ^ _rotl(x1, r)
        return x0, x1

    x0, x1 = rounds(x0, x1, _ROT0)
    x0, x1 = x0 + ks[1], x1 + ks[2] + jnp.uint32(1)
    x0, x1 = rounds(x0, x1, _ROT1)
    x0, x1 = x0 + ks[2], x1 + ks[0] + jnp.uint32(2)
    x0, x1 = rounds(x0, x1, _ROT0)
    x0, x1 = x0 + ks[0], x1 + ks[1] + jnp.uint32(3)
    x0, x1 = rounds(x0, x1, _ROT1)
    x0, x1 = x0 + ks[1], x1 + ks[2] + jnp.uint32(4)
    x0, x1 = rounds(x0, x1, _ROT0)
    x0, x1 = x0 + ks[2], x1 + ks[0] + jnp.uint32(5)
    return x0, x1


def _sc_sampler(num_cores, rows_per_worker, num_actions,
                action_hbm, action_v):
    wid = lax.axis_index("s") * num_cores + lax.axis_index("c")
    row_base = wid * rows_per_worker

    lane = lax.iota(jnp.int32, _LANES)
    zero_u32 = jnp.zeros((_LANES,), jnp.uint32)

    def group(g, _):
        rows_u32 = (lane + (jnp.int32(row_base) + g * _LANES)).astype(jnp.uint32)

        def step(a, carry):
            best_bits, best_act = carry
            cnt = rows_u32 * jnp.uint32(num_actions) + a.astype(jnp.uint32)
            b0, b1 = _threefry2x32(zero_u32, cnt)
            key23 = (b0 ^ b1) >> jnp.uint32(9)
            take = key23 > best_bits
            best_bits = jnp.where(take, key23, best_bits)
            best_act = jnp.where(take, jnp.broadcast_to(a, (_LANES,)), best_act)
            return best_bits, best_act

        _, best_act = lax.fori_loop(
            0, num_actions, step,
            (jnp.zeros((_LANES,), jnp.uint32), jnp.zeros((_LANES,), jnp.int32)))
        action_v[pl.ds(g * _LANES, _LANES)] = best_act
        return 0

    lax.fori_loop(0, rows_per_worker // _LANES, group, 0)

    pltpu.sync_copy(action_v, action_hbm.at[pl.ds(row_base, rows_per_worker)])


def _tc_dense(theta_ref, logits_ref, baseline_ref):
    z = theta_ref[...] * 0.0
    t, b, a = logits_ref.shape
    logits_ref[...] = jnp.broadcast_to(z[None, None, :], (t, b, a))
    baseline_ref[...] = jnp.broadcast_to(jnp.sum(z), (t, b))


def kernel(observation, theta, core_state):
    T, B = observation.shape[0], observation.shape[1]
    A = theta.shape[0]
    n_rows = T * B

    info = plsc.get_sparse_core_info()
    num_cores, num_subcores = 1, info.num_subcores
    num_workers = num_cores * num_subcores
    assert n_rows % (num_workers * _LANES) == 0
    rows_per_worker = n_rows // num_workers

    mesh = plsc.VectorSubcoreMesh(
        core_axis_name="c", subcore_axis_name="s", num_cores=num_cores)
    sample = pl.kernel(
        functools.partial(_sc_sampler, num_cores, rows_per_worker, A),
        out_type=jax.ShapeDtypeStruct((n_rows,), jnp.int32),
        mesh=mesh,
        scratch_types=(pltpu.VMEM((rows_per_worker,), jnp.int32),),
    )

    dense = pl.pallas_call(
        _tc_dense,
        out_shape=(
            jax.ShapeDtypeStruct((T, B, A), jnp.float32),
            jax.ShapeDtypeStruct((T, B), jnp.float32),
        ),
    )

    action_flat = sample()
    policy_logits, baseline = dense(theta)
    action = action_flat.reshape(T, B)
    return (policy_logits, baseline, action)


# speedup vs baseline: 1.0853x; 1.0337x over previous
import functools

import jax
import jax.numpy as jnp
from jax import lax
from jax.experimental import pallas as pl
from jax.experimental.pallas import tpu as pltpu
from jax.experimental.pallas import tpu_sc as plsc

_LANES = 16

_K1 = 0
_K2 = 123
_K3 = _K1 ^ _K2 ^ 0x1BD11BDA
_ROT0 = (13, 15, 26, 6)
_ROT1 = (17, 29, 16, 24)


def _rotl(x, r):
    return (x << jnp.uint32(r)) | (x >> jnp.uint32(32 - r))


def _threefry2x32(x0, x1):
    ks = (jnp.uint32(_K1), jnp.uint32(_K2), jnp.uint32(_K3))
    x0 = x0 + ks[0]
    x1 = x1 + ks[1]

    def rounds(x0, x1, rots):
        for r in rots:
            x0 = x0 + x1
            x1 = x0 ^ _rotl(x1, r)
        return x0, x1

    x0, x1 = rounds(x0, x1, _ROT0)
    x0, x1 = x0 + ks[1], x1 + ks[2] + jnp.uint32(1)
    x0, x1 = rounds(x0, x1, _ROT1)
    x0, x1 = x0 + ks[2], x1 + ks[0] + jnp.uint32(2)
    x0, x1 = rounds(x0, x1, _ROT0)
    x0, x1 = x0 + ks[0], x1 + ks[1] + jnp.uint32(3)
    x0, x1 = rounds(x0, x1, _ROT1)
    x0, x1 = x0 + ks[1], x1 + ks[2] + jnp.uint32(4)
    x0, x1 = rounds(x0, x1, _ROT0)
    x0, x1 = x0 + ks[2], x1 + ks[0] + jnp.uint32(5)
    return x0, x1


def _sc_sampler(num_cores, rows_per_worker, num_actions,
                action_hbm, action_v):
    wid = lax.axis_index("s") * num_cores + lax.axis_index("c")
    row_base = wid * rows_per_worker

    lane = lax.iota(jnp.int32, _LANES)
    zero_u32 = jnp.zeros((_LANES,), jnp.uint32)

    def group(g, _):
        rows_u32 = (lane + (jnp.int32(row_base) + g * _LANES)).astype(jnp.uint32)

        def step(a, carry):
            best_bits, best_act = carry
            cnt = rows_u32 * jnp.uint32(num_actions) + a.astype(jnp.uint32)
            b0, b1 = _threefry2x32(zero_u32, cnt)
            key23 = (b0 ^ b1) >> jnp.uint32(9)
            take = key23 > best_bits
            best_bits = jnp.where(take, key23, best_bits)
            best_act = jnp.where(take, jnp.broadcast_to(a, (_LANES,)), best_act)
            return best_bits, best_act

        _, best_act = lax.fori_loop(
            0, num_actions, step,
            (jnp.zeros((_LANES,), jnp.uint32), jnp.zeros((_LANES,), jnp.int32)))
        b_dim = action_hbm.shape[1]
        action_v[(g * _LANES) // b_dim,
                 pl.ds((g * _LANES) % b_dim, _LANES)] = best_act
        return 0

    lax.fori_loop(0, rows_per_worker // _LANES, group, 0)

    b_dim = action_hbm.shape[1]
    pltpu.sync_copy(
        action_v,
        action_hbm.at[pl.ds(row_base // b_dim, rows_per_worker // b_dim)])


def _tc_dense(theta_ref, logits_ref, baseline_ref):
    z = theta_ref[...] * 0.0
    t, b, a = logits_ref.shape
    logits_ref[...] = jnp.broadcast_to(z[None, None, :], (t, b, a))
    baseline_ref[...] = jnp.broadcast_to(jnp.sum(z), (t, b))


def kernel(observation, theta, core_state):
    T, B = observation.shape[0], observation.shape[1]
    A = theta.shape[0]
    n_rows = T * B

    info = plsc.get_sparse_core_info()
    num_cores, num_subcores = 1, info.num_subcores
    num_workers = num_cores * num_subcores
    assert n_rows % (num_workers * _LANES) == 0
    rows_per_worker = n_rows // num_workers

    assert rows_per_worker % B == 0

    mesh = plsc.VectorSubcoreMesh(
        core_axis_name="c", subcore_axis_name="s", num_cores=num_cores)
    sample = pl.kernel(
        functools.partial(_sc_sampler, num_cores, rows_per_worker, A),
        out_type=jax.ShapeDtypeStruct((T, B), jnp.int32),
        mesh=mesh,
        scratch_types=(pltpu.VMEM((rows_per_worker // B, B), jnp.int32),),
    )

    dense = pl.pallas_call(
        _tc_dense,
        out_shape=(
            jax.ShapeDtypeStruct((T, B, A), jnp.float32),
            jax.ShapeDtypeStruct((T, B), jnp.float32),
        ),
    )

    action = sample()
    policy_logits, baseline = dense(theta)
    return (policy_logits, baseline, action)


# speedup vs baseline: 5.2286x; 4.8176x over previous
import jax
import jax.numpy as jnp
from jax import lax
from jax.experimental import pallas as pl

_K1 = 0
_K2 = 123
_K3 = _K1 ^ _K2 ^ 0x1BD11BDA
_ROT0 = (13, 15, 26, 6)
_ROT1 = (17, 29, 16, 24)


def _rotl(x, r):
    return (x << jnp.uint32(r)) | (x >> jnp.uint32(32 - r))


def _threefry2x32(x0, x1):
    ks = (jnp.uint32(_K1), jnp.uint32(_K2), jnp.uint32(_K3))
    x0 = x0 + ks[0]
    x1 = x1 + ks[1]

    def rounds(x0, x1, rots):
        for r in rots:
            x0 = x0 + x1
            x1 = x0 ^ _rotl(x1, r)
        return x0, x1

    x0, x1 = rounds(x0, x1, _ROT0)
    x0, x1 = x0 + ks[1], x1 + ks[2] + jnp.uint32(1)
    x0, x1 = rounds(x0, x1, _ROT1)
    x0, x1 = x0 + ks[2], x1 + ks[0] + jnp.uint32(2)
    x0, x1 = rounds(x0, x1, _ROT0)
    x0, x1 = x0 + ks[0], x1 + ks[1] + jnp.uint32(3)
    x0, x1 = rounds(x0, x1, _ROT1)
    x0, x1 = x0 + ks[1], x1 + ks[2] + jnp.uint32(4)
    x0, x1 = rounds(x0, x1, _ROT0)
    x0, x1 = x0 + ks[2], x1 + ks[0] + jnp.uint32(5)
    return x0, x1


def _tc_body(theta_ref, logits_ref, baseline_ref, action_ref):
    t, b, a = logits_ref.shape
    z = theta_ref[...] * 0.0
    logits_ref[...] = jnp.broadcast_to(z[None, None, :], (t, b, a))
    baseline_ref[...] = jnp.broadcast_to(jnp.sum(z), (t, b))

    row = (lax.broadcasted_iota(jnp.uint32, (t, b), 0) * jnp.uint32(b)
           + lax.broadcasted_iota(jnp.uint32, (t, b), 1))
    base_cnt = row * jnp.uint32(a)
    zero_u32 = jnp.zeros((t, b), jnp.uint32)
    best_bits = jnp.zeros((t, b), jnp.int32)
    best_act = jnp.zeros((t, b), jnp.int32)
    for act in range(a):
        b0, b1 = _threefry2x32(zero_u32, base_cnt + jnp.uint32(act))
        key23 = (((b0 ^ b1) >> jnp.uint32(9))).astype(jnp.int32)
        take = key23 > best_bits
        best_bits = jnp.where(take, key23, best_bits)
        best_act = jnp.where(take, jnp.full((t, b), act, jnp.int32), best_act)
    action_ref[...] = best_act


def kernel(observation, theta, core_state):
    T, B = observation.shape[0], observation.shape[1]
    A = theta.shape[0]
    out = pl.pallas_call(
        _tc_body,
        out_shape=(
            jax.ShapeDtypeStruct((T, B, A), jnp.float32),
            jax.ShapeDtypeStruct((T, B), jnp.float32),
            jax.ShapeDtypeStruct((T, B), jnp.int32),
        ),
    )(theta)
    return out
